# baseline (speedup 1.0000x reference)
import functools

import jax
import jax.numpy as jnp
from jax import lax
from jax.experimental import pallas as pl
from jax.experimental.pallas import tpu as pltpu
from jax.experimental.pallas import tpu_sc as plsc

_N = 10000
_D = 128
_E = 320000

_NC = 2
_NS = 16
_NW = _NC * _NS
_NPAD = 10112
_RPT = _NPAD // _NS
_CHUNK = 128
_JW = 81
_JPRE = _JW + 3
_EPW = _JW * _CHUNK
_EPAD = _EPW * _NW
_GROWS = _JW * _NW


@functools.cache
def _get_sc_segment_sum():
    mesh = plsc.VectorSubcoreMesh(
        core_axis_name="c", subcore_axis_name="s",
        num_cores=_NC, num_subcores=_NS)
    return functools.partial(
        pl.kernel,
        mesh=mesh,
        out_type=jax.ShapeDtypeStruct((_NC, _NPAD, _D), jnp.float32),
        scratch_types=[
            pltpu.VMEM((2, _CHUNK), jnp.int32),
            pltpu.VMEM((2, _CHUNK), jnp.int32),
            pltpu.VMEM((2, _CHUNK), jnp.int32),
            pltpu.VMEM((_CHUNK, _D), jnp.float32),
            pltpu.VMEM((_CHUNK, _D), jnp.float32),
            pltpu.VMEM((_CHUNK, _D), jnp.float32),
            pltpu.VMEM_SHARED((_NPAD, _D), jnp.float32),
            pltpu.SemaphoreType.DMA,
            pltpu.SemaphoreType.DMA,
            pltpu.SemaphoreType.DMA,
            pltpu.SemaphoreType.DMA,
            pltpu.SemaphoreType.DMA,
            pltpu.SemaphoreType.DMA,
        ],
    )(_sc_segment_sum_body)


def _sc_segment_sum_body(idx_hbm, h_hbm, zeros_hbm, out_hbm,
                         idx0, idx1, idx2, rows0, rows1, rows2, acc,
                         isem0, isem1, isem2, gsem0, gsem1, gsem2):
    cid = lax.axis_index("c")
    sid = lax.axis_index("s")
    wid = sid * _NC + cid
    r0 = sid * _RPT

    idxb = (idx0, idx1, idx2)
    rows = (rows0, rows1, rows2)
    isem = (isem0, isem1, isem2)
    gsem = (gsem0, gsem1, gsem2)

    pltpu.sync_copy(zeros_hbm.at[pl.ds(r0, _RPT)], acc.at[pl.ds(r0, _RPT)])
    plsc.subcore_barrier()

    for u in range(3):
        pltpu.async_copy(idx_hbm.at[wid + _NW * u], idxb[u], isem[u])
    pltpu.make_async_copy(idx_hbm.at[wid], idx0, isem0).wait()
    pltpu.async_copy(h_hbm.at[idx0.at[0]], rows0, gsem0)

    def round_fn(r, carry):
        for u in range(3):
            c = 3 * r + u
            nb = (u + 1) % 3
            pltpu.make_async_copy(h_hbm.at[idx0.at[0]], rows[u],
                                  gsem[u]).wait()
            pltpu.make_async_copy(idx_hbm.at[wid], idxb[nb],
                                  isem[nb]).wait()
            pltpu.async_copy(h_hbm.at[idxb[nb].at[0]], rows[nb], gsem[nb])
            pltpu.sync_copy(rows[u], acc.at[idxb[u].at[1]], add=True)
            pltpu.async_copy(idx_hbm.at[wid + _NW * (c + 3)], idxb[u],
                             isem[u])
        return carry

    lax.fori_loop(0, _JW // 3, round_fn, 0)
    pltpu.make_async_copy(h_hbm.at[idx0.at[0]], rows0, gsem0).wait()
    pltpu.make_async_copy(idx_hbm.at[wid], idx1, isem1).wait()
    pltpu.make_async_copy(idx_hbm.at[wid], idx2, isem2).wait()
    plsc.subcore_barrier()

    pltpu.sync_copy(acc.at[pl.ds(r0, _RPT)],
                    out_hbm.at[cid, pl.ds(r0, _RPT), :])


def _bn(z, valid, g, b):
    zm = jnp.where(valid, z, 0.0)
    mean = jnp.sum(zm, axis=0, keepdims=True) * (1.0 / _N)
    var = jnp.sum(zm * zm, axis=0, keepdims=True) * (1.0 / _N) - mean * mean
    return (z - mean) * lax.rsqrt(var + 1e-5) * g + b


def _make_mlp(trailing_bn):
    def body(h_ref, p_ref, w1_ref, b1_ref, g1_ref, be1_ref, w2_ref, b2_ref,
             *rest):
        if trailing_bn:
            bng_ref, bnb_ref, out_ref = rest
        else:
            (out_ref,) = rest
        valid = lax.broadcasted_iota(jnp.int32, (_NPAD, 1), 0) < _N
        a = h_ref[...] + p_ref[0] + p_ref[1]
        a = jnp.where(valid, a, 0.0)
        z = jnp.dot(a, w1_ref[...], preferred_element_type=jnp.float32)
        z = z + b1_ref[...]
        z = _bn(z, valid, g1_ref[...], be1_ref[...])
        z = jnp.maximum(z, 0.0)
        z = jnp.dot(z, w2_ref[...], preferred_element_type=jnp.float32)
        z = z + b2_ref[...]
        if trailing_bn:
            z = _bn(z, valid, bng_ref[...], bnb_ref[...])
            z = jnp.maximum(z, 0.0)
        out_ref[...] = jnp.where(valid, z, 0.0)

    return pl.pallas_call(
        body,
        out_shape=jax.ShapeDtypeStruct((_NPAD, _D), jnp.float32),
    )


_mlp_mid = _make_mlp(True)
_mlp_final = _make_mlp(False)


def kernel(x, edge_index, params):
    pad = _EPAD - _E
    src = jnp.concatenate(
        [edge_index[0], jnp.full((pad,), _NPAD - 1, jnp.int32)])
    dst = jnp.concatenate(
        [edge_index[1], jnp.full((pad,), _NPAD - 1, jnp.int32)])
    idx = jnp.stack([src.reshape(_GROWS, _CHUNK),
                     dst.reshape(_GROWS, _CHUNK)], axis=1)
    idx = jnp.concatenate(
        [idx, jnp.full((_JPRE * _NW - _GROWS, 2, _CHUNK), _NPAD - 1,
                       jnp.int32)])
    zeros = jnp.zeros((_NPAD, _D), jnp.float32)
    h = jnp.zeros((_NPAD, _D), jnp.float32).at[:_N].set(x)

    sc_segment_sum = _get_sc_segment_sum()
    for i in (1, 2, 3):
        parts = sc_segment_sum(idx, h, zeros)
        args = (h, parts,
                params[f'W{i}_1'], params[f'b{i}_1'],
                params[f'mlp_g{i}'], params[f'mlp_b{i}'],
                params[f'W{i}_2'], params[f'b{i}_2'])
        if i < 3:
            h = _mlp_mid(*args, params[f'bn_g{i}'], params[f'bn_b{i}'])
        else:
            h = _mlp_final(*args)
    return h[:_N]


# speedup vs baseline: 4.6042x; 4.6042x over previous
import functools

import jax
import jax.numpy as jnp
from jax import lax
from jax.experimental import pallas as pl
from jax.experimental.pallas import tpu as pltpu
from jax.experimental.pallas import tpu_sc as plsc

_N = 10000
_D = 128
_E = 320000

_NC = 2
_NS = 16
_NW = _NC * _NS
_NPAD = 10112
_RPT = _NPAD // _NS
_CHUNK = 128
_JW = 81
_JPRE = _JW + 3
_EPW = _JW * _CHUNK
_EPAD = _EPW * _NW
_GROWS = _JW * _NW


@functools.cache
def _get_sc_segment_sum():
    mesh = plsc.VectorSubcoreMesh(
        core_axis_name="c", subcore_axis_name="s",
        num_cores=_NC, num_subcores=_NS)
    return functools.partial(
        pl.kernel,
        mesh=mesh,
        out_type=jax.ShapeDtypeStruct((_NC, _NPAD, _D), jnp.float32),
        scratch_types=[
            pltpu.VMEM((2, _CHUNK), jnp.int32),
            pltpu.VMEM((2, _CHUNK), jnp.int32),
            pltpu.VMEM((2, _CHUNK), jnp.int32),
            pltpu.VMEM((_CHUNK, _D), jnp.float32),
            pltpu.VMEM((_CHUNK, _D), jnp.float32),
            pltpu.VMEM((_CHUNK, _D), jnp.float32),
            pltpu.VMEM_SHARED((_NPAD, _D), jnp.float32),
            pltpu.SemaphoreType.DMA,
            pltpu.SemaphoreType.DMA,
            pltpu.SemaphoreType.DMA,
            pltpu.SemaphoreType.DMA,
            pltpu.SemaphoreType.DMA,
            pltpu.SemaphoreType.DMA,
        ],
    )(_sc_segment_sum_body)


def _sc_segment_sum_body(idx_hbm, h_hbm, zeros_hbm, out_hbm,
                         idx0, idx1, idx2, rows0, rows1, rows2, acc,
                         isem0, isem1, isem2, gsem0, gsem1, gsem2):
    cid = lax.axis_index("c")
    sid = lax.axis_index("s")
    wid = sid * _NC + cid
    r0 = sid * _RPT

    idxb = (idx0, idx1, idx2)
    rows = (rows0, rows1, rows2)
    isem = (isem0, isem1, isem2)
    gsem = (gsem0, gsem1, gsem2)

    pltpu.sync_copy(zeros_hbm.at[pl.ds(r0, _RPT)], acc.at[pl.ds(r0, _RPT)])
    plsc.subcore_barrier()

    for u in range(3):
        pltpu.async_copy(idx_hbm.at[wid + _NW * u], idxb[u], isem[u])
    pltpu.make_async_copy(idx_hbm.at[wid], idx0, isem0).wait()
    pltpu.async_copy(h_hbm.at[idx0.at[0]], rows0, gsem0)

    def round_fn(r, carry):
        for u in range(3):
            c = 3 * r + u
            nb = (u + 1) % 3
            pltpu.make_async_copy(h_hbm.at[idx0.at[0]], rows[u],
                                  gsem[u]).wait()
            pltpu.make_async_copy(idx_hbm.at[wid], idxb[nb],
                                  isem[nb]).wait()
            pltpu.async_copy(h_hbm.at[idxb[nb].at[0]], rows[nb], gsem[nb])
            pltpu.sync_copy(rows[u], acc.at[idxb[u].at[1]], add=True)
            pltpu.async_copy(idx_hbm.at[wid + _NW * (c + 3)], idxb[u],
                             isem[u])
        return carry

    lax.fori_loop(0, _JW // 3, round_fn, 0)
    pltpu.make_async_copy(h_hbm.at[idx0.at[0]], rows0, gsem0).wait()
    pltpu.make_async_copy(idx_hbm.at[wid], idx1, isem1).wait()
    pltpu.make_async_copy(idx_hbm.at[wid], idx2, isem2).wait()
    plsc.subcore_barrier()

    pltpu.sync_copy(acc.at[pl.ds(r0, _RPT)],
                    out_hbm.at[cid, pl.ds(r0, _RPT), :])


def _bn(z, valid, g, b):
    zm = jnp.where(valid, z, 0.0)
    mean = jnp.sum(zm, axis=0, keepdims=True) * (1.0 / _N)
    var = jnp.sum(zm * zm, axis=0, keepdims=True) * (1.0 / _N) - mean * mean
    return (z - mean) * lax.rsqrt(var + 1e-5) * g + b


def _make_mlp(trailing_bn):
    def body(h_ref, p_ref, w1_ref, b1_ref, g1_ref, be1_ref, w2_ref, b2_ref,
             *rest):
        if trailing_bn:
            bng_ref, bnb_ref, out_ref = rest
        else:
            (out_ref,) = rest
        valid = lax.broadcasted_iota(jnp.int32, (_NPAD, 1), 0) < _N
        a = h_ref[...] + p_ref[0] + p_ref[1]
        a = jnp.where(valid, a, 0.0)
        z = jnp.dot(a, w1_ref[...], preferred_element_type=jnp.float32)
        z = z + b1_ref[...]
        z = _bn(z, valid, g1_ref[...], be1_ref[...])
        z = jnp.maximum(z, 0.0)
        z = jnp.dot(z, w2_ref[...], preferred_element_type=jnp.float32)
        z = z + b2_ref[...]
        if trailing_bn:
            z = _bn(z, valid, bng_ref[...], bnb_ref[...])
            z = jnp.maximum(z, 0.0)
        out_ref[...] = jnp.where(valid, z, 0.0)

    return pl.pallas_call(
        body,
        out_shape=jax.ShapeDtypeStruct((_NPAD, _D), jnp.float32),
    )


_mlp_mid = _make_mlp(True)
_mlp_final = _make_mlp(False)


def kernel(x, edge_index, params):
    pad = _EPAD - _E
    fill = (jnp.arange(pad, dtype=jnp.int32) % (_NPAD - _N)) + _N
    src = jnp.concatenate([edge_index[0], fill])
    dst = jnp.concatenate([edge_index[1], fill])
    idx = jnp.stack([src.reshape(_GROWS, _CHUNK),
                     dst.reshape(_GROWS, _CHUNK)], axis=1)
    tail = (jnp.arange(_JPRE * _NW - _GROWS, dtype=jnp.int32)[:, None, None]
            + jnp.arange(2, dtype=jnp.int32)[None, :, None] * 7
            + jnp.arange(_CHUNK, dtype=jnp.int32)[None, None, :])
    idx = jnp.concatenate([idx, tail % (_NPAD - _N) + _N])
    zeros = jnp.zeros((_NPAD, _D), jnp.float32)
    h = jnp.zeros((_NPAD, _D), jnp.float32).at[:_N].set(x)

    sc_segment_sum = _get_sc_segment_sum()
    for i in (1, 2, 3):
        parts = sc_segment_sum(idx, h, zeros)
        args = (h, parts,
                params[f'W{i}_1'], params[f'b{i}_1'],
                params[f'mlp_g{i}'], params[f'mlp_b{i}'],
                params[f'W{i}_2'], params[f'b{i}_2'])
        if i < 3:
            h = _mlp_mid(*args, params[f'bn_g{i}'], params[f'bn_b{i}'])
        else:
            h = _mlp_final(*args)
    return h[:_N]


# speedup vs baseline: 4.8488x; 1.0531x over previous
import functools

import jax
import jax.numpy as jnp
from jax import lax
from jax.experimental import pallas as pl
from jax.experimental.pallas import tpu as pltpu
from jax.experimental.pallas import tpu_sc as plsc

_N = 10000
_D = 128
_E = 320000

_NC = 2
_NS = 16
_NW = _NC * _NS
_NPAD = 10112
_RPT = _NPAD // _NS
_CHUNK = 128
_JW = 81
_JPRE = _JW + 3
_EPW = _JW * _CHUNK
_EPAD = _EPW * _NW
_GROWS = _JW * _NW


@functools.cache
def _get_sc_segment_sum():
    mesh = plsc.VectorSubcoreMesh(
        core_axis_name="c", subcore_axis_name="s",
        num_cores=_NC, num_subcores=_NS)
    return functools.partial(
        pl.kernel,
        mesh=mesh,
        out_type=jax.ShapeDtypeStruct((_NC, _NPAD, _D), jnp.float32),
        scratch_types=[
            pltpu.VMEM((2, _CHUNK), jnp.int32),
            pltpu.VMEM((2, _CHUNK), jnp.int32),
            pltpu.VMEM((2, _CHUNK), jnp.int32),
            pltpu.VMEM((_CHUNK, _D), jnp.float32),
            pltpu.VMEM((_CHUNK, _D), jnp.float32),
            pltpu.VMEM((_CHUNK, _D), jnp.float32),
            pltpu.VMEM_SHARED((_NPAD, _D), jnp.float32),
            pltpu.SemaphoreType.DMA,
            pltpu.SemaphoreType.DMA,
            pltpu.SemaphoreType.DMA,
            pltpu.SemaphoreType.DMA,
            pltpu.SemaphoreType.DMA,
            pltpu.SemaphoreType.DMA,
        ],
    )(_sc_segment_sum_body)


def _sc_segment_sum_body(idx_hbm, h_hbm, zeros_hbm, out_hbm,
                         idx0, idx1, idx2, rows0, rows1, rows2, acc,
                         isem0, isem1, isem2, gsem0, gsem1, gsem2):
    cid = lax.axis_index("c")
    sid = lax.axis_index("s")
    wid = sid * _NC + cid
    r0 = sid * _RPT

    idxb = (idx0, idx1, idx2)
    rows = (rows0, rows1, rows2)
    isem = (isem0, isem1, isem2)
    gsem = (gsem0, gsem1, gsem2)

    pltpu.sync_copy(zeros_hbm.at[pl.ds(r0, _RPT)], acc.at[pl.ds(r0, _RPT)])
    plsc.subcore_barrier()

    for u in range(3):
        pltpu.async_copy(idx_hbm.at[wid + _NW * u], idxb[u], isem[u])
    pltpu.make_async_copy(idx_hbm.at[wid], idx0, isem0).wait()
    pltpu.async_copy(h_hbm.at[idx0.at[0]], rows0, gsem0)
    pltpu.make_async_copy(idx_hbm.at[wid], idx1, isem1).wait()
    pltpu.async_copy(h_hbm.at[idx1.at[0]], rows1, gsem1)

    def round_fn(r, carry):
        for u in range(3):
            c = 3 * r + u
            nb2 = (u + 2) % 3
            pltpu.make_async_copy(idx_hbm.at[wid], idxb[nb2],
                                  isem[nb2]).wait()
            pltpu.async_copy(h_hbm.at[idxb[nb2].at[0]], rows[nb2],
                             gsem[nb2])
            pltpu.make_async_copy(h_hbm.at[idx0.at[0]], rows[u],
                                  gsem[u]).wait()
            pltpu.sync_copy(rows[u], acc.at[idxb[u].at[1]], add=True)
            pltpu.async_copy(idx_hbm.at[wid + _NW * (c + 3)], idxb[u],
                             isem[u])
        return carry

    lax.fori_loop(0, _JW // 3, round_fn, 0)
    pltpu.make_async_copy(h_hbm.at[idx0.at[0]], rows0, gsem0).wait()
    pltpu.make_async_copy(h_hbm.at[idx0.at[0]], rows1, gsem1).wait()
    pltpu.make_async_copy(idx_hbm.at[wid], idx2, isem2).wait()
    plsc.subcore_barrier()

    pltpu.sync_copy(acc.at[pl.ds(r0, _RPT)],
                    out_hbm.at[cid, pl.ds(r0, _RPT), :])


def _bn(z, valid, g, b):
    zm = jnp.where(valid, z, 0.0)
    mean = jnp.sum(zm, axis=0, keepdims=True) * (1.0 / _N)
    var = jnp.sum(zm * zm, axis=0, keepdims=True) * (1.0 / _N) - mean * mean
    return (z - mean) * lax.rsqrt(var + 1e-5) * g + b


def _make_mlp(trailing_bn):
    def body(h_ref, p_ref, w1_ref, b1_ref, g1_ref, be1_ref, w2_ref, b2_ref,
             *rest):
        if trailing_bn:
            bng_ref, bnb_ref, out_ref = rest
        else:
            (out_ref,) = rest
        valid = lax.broadcasted_iota(jnp.int32, (_NPAD, 1), 0) < _N
        a = h_ref[...] + p_ref[0] + p_ref[1]
        a = jnp.where(valid, a, 0.0)
        z = jnp.dot(a, w1_ref[...], preferred_element_type=jnp.float32)
        z = z + b1_ref[...]
        z = _bn(z, valid, g1_ref[...], be1_ref[...])
        z = jnp.maximum(z, 0.0)
        z = jnp.dot(z, w2_ref[...], preferred_element_type=jnp.float32)
        z = z + b2_ref[...]
        if trailing_bn:
            z = _bn(z, valid, bng_ref[...], bnb_ref[...])
            z = jnp.maximum(z, 0.0)
        out_ref[...] = jnp.where(valid, z, 0.0)

    return pl.pallas_call(
        body,
        out_shape=jax.ShapeDtypeStruct((_NPAD, _D), jnp.float32),
    )


_mlp_mid = _make_mlp(True)
_mlp_final = _make_mlp(False)


def kernel(x, edge_index, params):
    pad = _EPAD - _E
    fill = (jnp.arange(pad, dtype=jnp.int32) % (_NPAD - _N)) + _N
    src = jnp.concatenate([edge_index[0], fill])
    dst = jnp.concatenate([edge_index[1], fill])
    idx = jnp.stack([src.reshape(_GROWS, _CHUNK),
                     dst.reshape(_GROWS, _CHUNK)], axis=1)
    tail = (jnp.arange(_JPRE * _NW - _GROWS, dtype=jnp.int32)[:, None, None]
            + jnp.arange(2, dtype=jnp.int32)[None, :, None] * 7
            + jnp.arange(_CHUNK, dtype=jnp.int32)[None, None, :])
    idx = jnp.concatenate([idx, tail % (_NPAD - _N) + _N])
    zeros = jnp.zeros((_NPAD, _D), jnp.float32)
    h = jnp.zeros((_NPAD, _D), jnp.float32).at[:_N].set(x)

    sc_segment_sum = _get_sc_segment_sum()
    for i in (1, 2, 3):
        parts = sc_segment_sum(idx, h, zeros)
        args = (h, parts,
                params[f'W{i}_1'], params[f'b{i}_1'],
                params[f'mlp_g{i}'], params[f'mlp_b{i}'],
                params[f'W{i}_2'], params[f'b{i}_2'])
        if i < 3:
            h = _mlp_mid(*args, params[f'bn_g{i}'], params[f'bn_b{i}'])
        else:
            h = _mlp_final(*args)
    return h[:_N]


# speedup vs baseline: 5.3698x; 1.1075x over previous
import functools

import jax
import jax.numpy as jnp
from jax import lax
from jax.experimental import pallas as pl
from jax.experimental.pallas import tpu as pltpu
from jax.experimental.pallas import tpu_sc as plsc

_N = 10000
_D = 128
_E = 320000

_NC = 2
_NS = 16
_NW = _NC * _NS
_NPAD = 10112
_RPT = _NPAD // _NS
_CHUNK = 128
_JW = 84
_JPRE = _JW + 3
_EPW = _JW * _CHUNK
_EPAD = _EPW * _NW
_GROWS = _EPAD // _CHUNK
_ROWB = _CHUNK * _D * 4


@functools.cache
def _get_sc_segment_sum(tbl_rows):
    mesh = plsc.VectorSubcoreMesh(
        core_axis_name="c", subcore_axis_name="s",
        num_cores=_NC, num_subcores=_NS)
    body = functools.partial(_sc_segment_sum_body, tbl_rows)
    return functools.partial(
        pl.kernel,
        mesh=mesh,
        out_type=jax.ShapeDtypeStruct((_NC, _NPAD, _D), jnp.float32),
        scratch_types=[
            pltpu.VMEM((_CHUNK,), jnp.int32),
            pltpu.VMEM((_CHUNK,), jnp.int32),
            pltpu.VMEM((_CHUNK,), jnp.int32),
            pltpu.VMEM((_CHUNK,), jnp.int32),
            pltpu.VMEM((_CHUNK,), jnp.int32),
            pltpu.VMEM((_CHUNK,), jnp.int32),
            pltpu.VMEM((_CHUNK,), jnp.int32),
            pltpu.VMEM((_CHUNK, _D), jnp.float32),
            pltpu.VMEM((_CHUNK, _D), jnp.float32),
            pltpu.VMEM((_CHUNK, _D), jnp.float32),
            pltpu.VMEM_SHARED((_NPAD, _D), jnp.float32),
            pltpu.SemaphoreType.DMA,
            pltpu.SemaphoreType.DMA,
            pltpu.SemaphoreType.DMA,
            pltpu.SemaphoreType.DMA,
            pltpu.SemaphoreType.DMA,
            pltpu.SemaphoreType.DMA,
            pltpu.SemaphoreType.DMA,
            pltpu.SemaphoreType.DMA,
            pltpu.SemaphoreType.DMA,
            pltpu.SemaphoreType.DMA,
            pltpu.SemaphoreType.DMA,
            pltpu.SemaphoreType.DMA,
            pltpu.SemaphoreType.DMA,
        ],
    )(body)


def _sc_segment_sum_body(tbl_rows, src_hbm, dst_hbm, h_hbm, zeros_hbm,
                         out_hbm,
                         srci0, srci1, srci2, dsti0, dsti1, dsti2, dsti3,
                         rows0, rows1, rows2, acc,
                         isem0, isem1, isem2, dsem0, dsem1, dsem2, dsem3,
                         gsem0, gsem1, gsem2, ssem0, ssem1, ssem2):
    del tbl_rows
    cid = lax.axis_index("c")
    sid = lax.axis_index("s")
    wid = sid * _NC + cid
    r0 = sid * _RPT

    srci = (srci0, srci1, srci2)
    dsti = (dsti0, dsti1, dsti2, dsti3)
    rows = (rows0, rows1, rows2)
    isem = (isem0, isem1, isem2)
    dsem = (dsem0, dsem1, dsem2, dsem3)
    gsem = (gsem0, gsem1, gsem2)
    ssem = (ssem0, ssem1, ssem2)

    pltpu.sync_copy(zeros_hbm.at[pl.ds(r0, _RPT)], acc.at[pl.ds(r0, _RPT)])
    plsc.subcore_barrier()

    for u in range(3):
        pltpu.async_copy(src_hbm.at[wid + _NW * u], srci[u], isem[u])
    pltpu.async_copy(dst_hbm.at[wid], dsti0, dsem0)
    pltpu.async_copy(dst_hbm.at[wid + _NW], dsti1, dsem1)
    pltpu.make_async_copy(src_hbm.at[wid], srci0, isem0).wait()
    pltpu.async_copy(h_hbm.at[srci0], rows0, gsem0)
    pltpu.sync_copy(dst_hbm.at[_GROWS], dsti2)
    pltpu.sync_copy(dst_hbm.at[_GROWS + 1], dsti3)
    pltpu.async_copy(rows1, acc.at[dsti2], ssem1, add=True)
    pltpu.async_copy(rows2, acc.at[dsti3], ssem2, add=True)

    def round_fn(r, carry):
        for p in range(12):
            c = 12 * r + p
            u3 = p % 3
            u4 = p % 4
            f3 = (p + 1) % 3
            f4 = (p + 2) % 4
            pltpu.make_async_copy(rows[f3], acc.at[dsti[u4]],
                                  ssem[f3]).wait()
            pltpu.async_copy(dst_hbm.at[wid + _NW * (c + 2)], dsti[f4],
                             dsem[f4])
            pltpu.make_async_copy(src_hbm.at[wid], srci[f3],
                                  isem[f3]).wait()
            pltpu.async_copy(h_hbm.at[srci[f3]], rows[f3], gsem[f3])
            pltpu.make_async_copy(h_hbm.at[srci0], rows[u3],
                                  gsem[u3]).wait()
            pltpu.make_async_copy(dst_hbm.at[wid], dsti[u4],
                                  dsem[u4]).wait()
            pltpu.async_copy(rows[u3], acc.at[dsti[u4]], ssem[u3],
                             add=True)
            pltpu.async_copy(src_hbm.at[wid + _NW * (c + 3)], srci[u3],
                             isem[u3])
        return carry

    lax.fori_loop(0, _JW // 12, round_fn, 0)
    pltpu.make_async_copy(h_hbm.at[srci0], rows0, gsem0).wait()
    pltpu.make_async_copy(rows1, acc.at[dsti1], ssem1).wait()
    pltpu.make_async_copy(rows2, acc.at[dsti2], ssem2).wait()
    pltpu.make_async_copy(src_hbm.at[wid], srci1, isem1).wait()
    pltpu.make_async_copy(src_hbm.at[wid], srci2, isem2).wait()
    pltpu.make_async_copy(dst_hbm.at[wid], dsti0, dsem0).wait()
    pltpu.make_async_copy(dst_hbm.at[wid], dsti1, dsem1).wait()
    plsc.subcore_barrier()

    pltpu.sync_copy(acc.at[pl.ds(r0, _RPT)],
                    out_hbm.at[cid, pl.ds(r0, _RPT), :])


def _bn(z, valid, g, b):
    zm = jnp.where(valid, z, 0.0)
    mean = jnp.sum(zm, axis=0, keepdims=True) * (1.0 / _N)
    var = jnp.sum(zm * zm, axis=0, keepdims=True) * (1.0 / _N) - mean * mean
    return (z - mean) * lax.rsqrt(var + 1e-5) * g + b


def _make_mlp(in_rows, out_rows, trailing_bn):
    def body(h_ref, p_ref, w1_ref, b1_ref, g1_ref, be1_ref, w2_ref, b2_ref,
             *rest):
        if trailing_bn:
            bng_ref, bnb_ref, out_ref = rest
        else:
            (out_ref,) = rest
        valid = lax.broadcasted_iota(jnp.int32, (_NPAD, 1), 0) < _N
        hv = h_ref[...]
        if in_rows < _NPAD:
            hv = jnp.concatenate(
                [hv, jnp.zeros((_NPAD - in_rows, _D), jnp.float32)], axis=0)
        a = hv + p_ref[0] + p_ref[1]
        a = jnp.where(valid, a, 0.0)
        z = jnp.dot(a, w1_ref[...], preferred_element_type=jnp.float32)
        z = z + b1_ref[...]
        z = _bn(z, valid, g1_ref[...], be1_ref[...])
        z = jnp.maximum(z, 0.0)
        z = jnp.dot(z, w2_ref[...], preferred_element_type=jnp.float32)
        z = z + b2_ref[...]
        if trailing_bn:
            z = _bn(z, valid, bng_ref[...], bnb_ref[...])
            z = jnp.maximum(z, 0.0)
        if out_rows < _NPAD:
            out_ref[...] = z[:out_rows]
        else:
            out_ref[...] = jnp.where(valid, z, 0.0)

    return pl.pallas_call(
        body,
        out_shape=jax.ShapeDtypeStruct((out_rows, _D), jnp.float32),
    )


_mlp_first = _make_mlp(_N, _NPAD, True)
_mlp_mid = _make_mlp(_NPAD, _NPAD, True)
_mlp_final = _make_mlp(_NPAD, _N, False)


def kernel(x, edge_index, params):
    pad = _EPAD - _E
    fs = jnp.arange(pad, dtype=jnp.int32) % _CHUNK
    fd = jnp.arange(pad, dtype=jnp.int32) % (_NPAD - _N) + _N
    tail_s = (jnp.arange(_JPRE * _NW - _GROWS, dtype=jnp.int32)[:, None]
              + jnp.arange(_CHUNK, dtype=jnp.int32)[None, :]) % _CHUNK
    tail_d = tail_s % (_NPAD - _N) + _N
    src = jnp.concatenate(
        [jnp.concatenate([edge_index[0], fs]).reshape(_GROWS, _CHUNK),
         tail_s])
    dst = jnp.concatenate(
        [jnp.concatenate([edge_index[1], fd]).reshape(_GROWS, _CHUNK),
         tail_d])
    zeros = jnp.zeros((_NPAD, _D), jnp.float32)

    h = x
    for i in (1, 2, 3):
        parts = _get_sc_segment_sum(h.shape[0])(src, dst, h, zeros)
        args = (h, parts,
                params[f'W{i}_1'], params[f'b{i}_1'],
                params[f'mlp_g{i}'], params[f'mlp_b{i}'],
                params[f'W{i}_2'], params[f'b{i}_2'])
        if i == 1:
            h = _mlp_first(*args, params[f'bn_g{i}'], params[f'bn_b{i}'])
        elif i == 2:
            h = _mlp_mid(*args, params[f'bn_g{i}'], params[f'bn_b{i}'])
        else:
            h = _mlp_final(*args)
    return h


# speedup vs baseline: 5.5161x; 1.0272x over previous
import functools

import jax
import jax.numpy as jnp
from jax import lax
from jax.experimental import pallas as pl
from jax.experimental.pallas import tpu as pltpu
from jax.experimental.pallas import tpu_sc as plsc

_N = 10000
_D = 128
_E = 320000

_NC = 2
_NS = 16
_NW = _NC * _NS
_NPAD = 10112
_RPT = _NPAD // _NS
_CHUNK = 128
_JW = 84
_JPRE = _JW + 3
_EPW = _JW * _CHUNK
_EPAD = _EPW * _NW
_GROWS = _EPAD // _CHUNK
_ROWB = _CHUNK * _D * 4


@functools.cache
def _get_sc_segment_sum(tbl_rows):
    mesh = plsc.VectorSubcoreMesh(
        core_axis_name="c", subcore_axis_name="s",
        num_cores=_NC, num_subcores=_NS)
    body = functools.partial(_sc_segment_sum_body, tbl_rows)
    return functools.partial(
        pl.kernel,
        mesh=mesh,
        out_type=jax.ShapeDtypeStruct((_NC, _NPAD, _D), jnp.float32),
        scratch_types=[
            pltpu.VMEM((_CHUNK,), jnp.int32),
            pltpu.VMEM((_CHUNK,), jnp.int32),
            pltpu.VMEM((_CHUNK,), jnp.int32),
            pltpu.VMEM((_CHUNK,), jnp.int32),
            pltpu.VMEM((_CHUNK,), jnp.int32),
            pltpu.VMEM((_CHUNK,), jnp.int32),
            pltpu.VMEM((_CHUNK,), jnp.int32),
            pltpu.VMEM((_CHUNK, _D), jnp.float32),
            pltpu.VMEM((_CHUNK, _D), jnp.float32),
            pltpu.VMEM((_CHUNK, _D), jnp.float32),
            pltpu.VMEM_SHARED((_NPAD, _D), jnp.float32),
            pltpu.SemaphoreType.DMA,
            pltpu.SemaphoreType.DMA,
            pltpu.SemaphoreType.DMA,
            pltpu.SemaphoreType.DMA,
            pltpu.SemaphoreType.DMA,
            pltpu.SemaphoreType.DMA,
            pltpu.SemaphoreType.DMA,
            pltpu.SemaphoreType.DMA,
            pltpu.SemaphoreType.DMA,
            pltpu.SemaphoreType.DMA,
            pltpu.SemaphoreType.DMA,
            pltpu.SemaphoreType.DMA,
            pltpu.SemaphoreType.DMA,
        ],
    )(body)


def _sc_segment_sum_body(tbl_rows, edges_hbm, h_hbm, zeros_hbm,
                         out_hbm,
                         srci0, srci1, srci2, dsti0, dsti1, dsti2, dsti3,
                         rows0, rows1, rows2, acc,
                         isem0, isem1, isem2, dsem0, dsem1, dsem2, dsem3,
                         gsem0, gsem1, gsem2, ssem0, ssem1, ssem2):
    del tbl_rows
    cid = lax.axis_index("c")
    sid = lax.axis_index("s")
    wid = sid * _NC + cid
    r0 = sid * _RPT

    srci = (srci0, srci1, srci2)
    dsti = (dsti0, dsti1, dsti2, dsti3)
    rows = (rows0, rows1, rows2)
    isem = (isem0, isem1, isem2)
    dsem = (dsem0, dsem1, dsem2, dsem3)
    gsem = (gsem0, gsem1, gsem2)
    ssem = (ssem0, ssem1, ssem2)

    def src_row(j):
        return edges_hbm.at[0, pl.ds((wid + _NW * j) * _CHUNK, _CHUNK)]

    def dst_row(j):
        return edges_hbm.at[1, pl.ds((wid + _NW * j) * _CHUNK, _CHUNK)]

    for u in range(3):
        pltpu.async_copy(src_row(u), srci[u], isem[u])
    pltpu.async_copy(dst_row(0), dsti0, dsem0)
    pltpu.async_copy(dst_row(1), dsti1, dsem1)
    pltpu.make_async_copy(src_row(0), srci0, isem0).wait()
    pltpu.async_copy(h_hbm.at[srci0], rows0, gsem0)
    pltpu.make_async_copy(src_row(0), srci1, isem1).wait()
    pltpu.async_copy(h_hbm.at[srci1], rows1, gsem1)
    pltpu.sync_copy(edges_hbm.at[1, pl.ds(_E, _CHUNK)], dsti3)
    pltpu.async_copy(rows2, acc.at[dsti3], ssem2, add=True)

    pltpu.sync_copy(zeros_hbm.at[pl.ds(r0, _RPT)], acc.at[pl.ds(r0, _RPT)])
    plsc.subcore_barrier()

    def round_fn(r, carry):
        for p in range(12):
            c = 12 * r + p
            u3 = p % 3
            u4 = p % 4
            f3 = (p + 2) % 3
            f4 = (p + 2) % 4
            pltpu.make_async_copy(rows[f3], acc.at[dsti[u4]],
                                  ssem[f3]).wait()
            pltpu.async_copy(dst_row(c + 2), dsti[f4], dsem[f4])
            pltpu.make_async_copy(src_row(0), srci[f3], isem[f3]).wait()
            pltpu.async_copy(h_hbm.at[srci[f3]], rows[f3], gsem[f3])
            pltpu.make_async_copy(h_hbm.at[srci0], rows[u3],
                                  gsem[u3]).wait()
            pltpu.make_async_copy(dst_row(0), dsti[u4], dsem[u4]).wait()
            pltpu.async_copy(rows[u3], acc.at[dsti[u4]], ssem[u3],
                             add=True)
            pltpu.async_copy(src_row(c + 3), srci[u3], isem[u3])
        return carry

    lax.fori_loop(0, _JW // 12, round_fn, 0)
    pltpu.make_async_copy(h_hbm.at[srci0], rows0, gsem0).wait()
    pltpu.make_async_copy(h_hbm.at[srci0], rows1, gsem1).wait()
    pltpu.make_async_copy(rows2, acc.at[dsti2], ssem2).wait()
    pltpu.make_async_copy(src_row(0), srci2, isem2).wait()
    pltpu.make_async_copy(dst_row(0), dsti0, dsem0).wait()
    pltpu.make_async_copy(dst_row(0), dsti1, dsem1).wait()
    plsc.subcore_barrier()

    pltpu.sync_copy(acc.at[pl.ds(r0, _RPT)],
                    out_hbm.at[cid, pl.ds(r0, _RPT), :])


def _bn(z, valid, g, b):
    zm = jnp.where(valid, z, 0.0)
    mean = jnp.sum(zm, axis=0, keepdims=True) * (1.0 / _N)
    var = jnp.sum(zm * zm, axis=0, keepdims=True) * (1.0 / _N) - mean * mean
    return (z - mean) * lax.rsqrt(var + 1e-5) * g + b


def _make_mlp(in_rows, out_rows, trailing_bn):
    def body(h_ref, p_ref, w1_ref, b1_ref, g1_ref, be1_ref, w2_ref, b2_ref,
             *rest):
        if trailing_bn:
            bng_ref, bnb_ref, out_ref = rest
        else:
            (out_ref,) = rest
        valid = lax.broadcasted_iota(jnp.int32, (_NPAD, 1), 0) < _N
        hv = h_ref[...]
        if in_rows < _NPAD:
            hv = jnp.concatenate(
                [hv, jnp.zeros((_NPAD - in_rows, _D), jnp.float32)], axis=0)
        a = hv + p_ref[0] + p_ref[1]
        a = jnp.where(valid, a, 0.0)
        z = jnp.dot(a, w1_ref[...], preferred_element_type=jnp.float32)
        z = z + b1_ref[...]
        z = _bn(z, valid, g1_ref[...], be1_ref[...])
        z = jnp.maximum(z, 0.0)
        z = jnp.dot(z, w2_ref[...], preferred_element_type=jnp.float32)
        z = z + b2_ref[...]
        if trailing_bn:
            z = _bn(z, valid, bng_ref[...], bnb_ref[...])
            z = jnp.maximum(z, 0.0)
        if out_rows < _NPAD:
            out_ref[...] = z[:out_rows]
        else:
            out_ref[...] = jnp.where(valid, z, 0.0)

    return pl.pallas_call(
        body,
        out_shape=jax.ShapeDtypeStruct((out_rows, _D), jnp.float32),
    )


_mlp_first = _make_mlp(_N, _NPAD, True)
_mlp_mid = _make_mlp(_NPAD, _NPAD, True)
_mlp_final = _make_mlp(_NPAD, _N, False)


def kernel(x, edge_index, params):
    pad = _JPRE * _NW * _CHUNK - _E
    ar = jnp.arange(pad, dtype=jnp.int32)
    edges = jnp.concatenate(
        [edge_index, jnp.stack([ar % _CHUNK, ar % (_NPAD - _N) + _N])],
        axis=1)
    zeros = jnp.zeros((_NPAD, _D), jnp.float32)

    h = x
    for i in (1, 2, 3):
        parts = _get_sc_segment_sum(h.shape[0])(edges, h, zeros)
        args = (h, parts,
                params[f'W{i}_1'], params[f'b{i}_1'],
                params[f'mlp_g{i}'], params[f'mlp_b{i}'],
                params[f'W{i}_2'], params[f'b{i}_2'])
        if i == 1:
            h = _mlp_first(*args, params[f'bn_g{i}'], params[f'bn_b{i}'])
        elif i == 2:
            h = _mlp_mid(*args, params[f'bn_g{i}'], params[f'bn_b{i}'])
        else:
            h = _mlp_final(*args)
    return h


# speedup vs baseline: 6.1108x; 1.1078x over previous
import functools

import numpy as np

import jax
import jax.numpy as jnp
from jax import lax
from jax.experimental import pallas as pl
from jax.experimental.pallas import tpu as pltpu
from jax.experimental.pallas import tpu_sc as plsc

_N = 10000
_D = 128
_E = 320000

_NC = 2
_NS = 16
_NW = _NC * _NS
_NPAD = 10112
_RPT = _NPAD // _NS
_CHUNK = 128
_JW = 81
_JPRE = _JW + 3
_EPW = _JW * _CHUNK
_EPAD = _EPW * _NW
_GROWS = _EPAD // _CHUNK
_ROWB = _CHUNK * _D * 4


@functools.cache
def _get_sc_segment_sum(tbl_rows):
    mesh = plsc.VectorSubcoreMesh(
        core_axis_name="c", subcore_axis_name="s",
        num_cores=_NC, num_subcores=_NS)
    body = functools.partial(_sc_segment_sum_body, tbl_rows)
    return functools.partial(
        pl.kernel,
        mesh=mesh,
        out_type=jax.ShapeDtypeStruct((_NC, _NPAD, _D), jnp.float32),
        scratch_types=[
            pltpu.VMEM((_CHUNK,), jnp.int32),
            pltpu.VMEM((_CHUNK,), jnp.int32),
            pltpu.VMEM((_CHUNK,), jnp.int32),
            pltpu.VMEM((_CHUNK,), jnp.int32),
            pltpu.VMEM((_CHUNK,), jnp.int32),
            pltpu.VMEM((_CHUNK,), jnp.int32),
            pltpu.VMEM((_CHUNK,), jnp.int32),
            pltpu.VMEM((_CHUNK, _D), jnp.float32),
            pltpu.VMEM((_CHUNK, _D), jnp.float32),
            pltpu.VMEM((_CHUNK, _D), jnp.float32),
            pltpu.VMEM_SHARED((_NPAD, _D), jnp.float32),
            pltpu.SemaphoreType.DMA,
            pltpu.SemaphoreType.DMA,
            pltpu.SemaphoreType.DMA,
            pltpu.SemaphoreType.DMA,
            pltpu.SemaphoreType.DMA,
            pltpu.SemaphoreType.DMA,
            pltpu.SemaphoreType.DMA,
            pltpu.SemaphoreType.DMA,
            pltpu.SemaphoreType.DMA,
            pltpu.SemaphoreType.DMA,
            pltpu.SemaphoreType.DMA,
            pltpu.SemaphoreType.DMA,
            pltpu.SemaphoreType.DMA,
        ],
    )(body)


def _sc_segment_sum_body(tbl_rows, edges_hbm, h_hbm, zeros_hbm,
                         out_hbm,
                         srci0, srci1, srci2, dsti0, dsti1, dsti2, dsti3,
                         rows0, rows1, rows2, acc,
                         isem0, isem1, isem2, dsem0, dsem1, dsem2, dsem3,
                         gsem0, gsem1, gsem2, ssem0, ssem1, ssem2):
    del tbl_rows
    cid = lax.axis_index("c")
    sid = lax.axis_index("s")
    wid = sid * _NC + cid
    r0 = sid * _RPT

    srci = (srci0, srci1, srci2)
    dsti = (dsti0, dsti1, dsti2, dsti3)
    rows = (rows0, rows1, rows2)
    isem = (isem0, isem1, isem2)
    dsem = (dsem0, dsem1, dsem2, dsem3)
    gsem = (gsem0, gsem1, gsem2)
    ssem = (ssem0, ssem1, ssem2)

    def src_row(j):
        return edges_hbm.at[0, pl.ds((wid + _NW * j) * _CHUNK, _CHUNK)]

    def dst_row(j):
        return edges_hbm.at[1, pl.ds((wid + _NW * j) * _CHUNK, _CHUNK)]

    for u in range(3):
        pltpu.async_copy(src_row(u), srci[u], isem[u])
    pltpu.async_copy(dst_row(0), dsti0, dsem0)
    pltpu.async_copy(dst_row(1), dsti1, dsem1)
    pltpu.make_async_copy(src_row(0), srci0, isem0).wait()
    pltpu.async_copy(h_hbm.at[srci0], rows0, gsem0)
    pltpu.make_async_copy(src_row(0), srci1, isem1).wait()
    pltpu.async_copy(h_hbm.at[srci1], rows1, gsem1)
    pltpu.sync_copy(edges_hbm.at[1, pl.ds(_E, _CHUNK)], dsti3)
    pltpu.async_copy(rows2, acc.at[dsti3], ssem2, add=True)

    pltpu.sync_copy(zeros_hbm.at[pl.ds(r0, _RPT)], acc.at[pl.ds(r0, _RPT)])
    plsc.subcore_barrier()

    def round_fn(r, carry):
        for p in range(3):
            c = 3 * r + p
            u3 = p
            f3 = (p + 2) % 3
            pltpu.make_async_copy(rows[f3], acc.at[dsti[u3]],
                                  ssem[f3]).wait()
            pltpu.async_copy(dst_row(c + 2), dsti[f3], dsem[f3])
            pltpu.make_async_copy(src_row(0), srci[f3], isem[f3]).wait()
            pltpu.async_copy(h_hbm.at[srci[f3]], rows[f3], gsem[f3])
            pltpu.make_async_copy(h_hbm.at[srci0], rows[u3],
                                  gsem[u3]).wait()
            pltpu.make_async_copy(dst_row(0), dsti[u3], dsem[u3]).wait()
            pltpu.async_copy(rows[u3], acc.at[dsti[u3]], ssem[u3],
                             add=True)
            pltpu.async_copy(src_row(c + 3), srci[u3], isem[u3])
        return carry

    lax.fori_loop(0, _JW // 3, round_fn, 0)
    pltpu.make_async_copy(h_hbm.at[srci0], rows0, gsem0).wait()
    pltpu.make_async_copy(h_hbm.at[srci0], rows1, gsem1).wait()
    pltpu.make_async_copy(rows2, acc.at[dsti2], ssem2).wait()
    pltpu.make_async_copy(src_row(0), srci2, isem2).wait()
    pltpu.make_async_copy(dst_row(0), dsti0, dsem0).wait()
    pltpu.make_async_copy(dst_row(0), dsti1, dsem1).wait()
    plsc.subcore_barrier()

    pltpu.sync_copy(acc.at[pl.ds(r0, _RPT)],
                    out_hbm.at[cid, pl.ds(r0, _RPT), :])


def _bn(z, valid, g, b):
    zm = jnp.where(valid, z, 0.0)
    mean = jnp.sum(zm, axis=0, keepdims=True) * (1.0 / _N)
    var = jnp.sum(zm * zm, axis=0, keepdims=True) * (1.0 / _N) - mean * mean
    return (z - mean) * lax.rsqrt(var + 1e-5) * g + b


def _make_mlp(in_rows, out_rows, trailing_bn):
    def body(h_ref, p_ref, w1_ref, b1_ref, g1_ref, be1_ref, w2_ref, b2_ref,
             *rest):
        if trailing_bn:
            bng_ref, bnb_ref, out_ref = rest
        else:
            (out_ref,) = rest
        valid = lax.broadcasted_iota(jnp.int32, (_NPAD, 1), 0) < _N
        hv = h_ref[...]
        if in_rows < _NPAD:
            hv = jnp.concatenate(
                [hv, jnp.zeros((_NPAD - in_rows, _D), jnp.float32)], axis=0)
        a = hv + p_ref[0] + p_ref[1]
        a = jnp.where(valid, a, 0.0)
        z = jnp.dot(a, w1_ref[...], preferred_element_type=jnp.float32)
        z = z + b1_ref[...]
        z = _bn(z, valid, g1_ref[...], be1_ref[...])
        z = jnp.maximum(z, 0.0)
        z = jnp.dot(z, w2_ref[...], preferred_element_type=jnp.float32)
        z = z + b2_ref[...]
        if trailing_bn:
            z = _bn(z, valid, bng_ref[...], bnb_ref[...])
            z = jnp.maximum(z, 0.0)
        if out_rows < _NPAD:
            out_ref[...] = z[:out_rows]
        else:
            out_ref[...] = jnp.where(valid, z, 0.0)

    return pl.pallas_call(
        body,
        out_shape=jax.ShapeDtypeStruct((out_rows, _D), jnp.float32),
    )


_mlp_first = _make_mlp(_N, _NPAD, True)
_mlp_mid = _make_mlp(_NPAD, _NPAD, True)
_mlp_final = _make_mlp(_NPAD, _N, False)


def kernel(x, edge_index, params):
    pad = _JPRE * _NW * _CHUNK - _E
    ar = np.arange(pad, dtype=np.int32)
    edges = jnp.concatenate(
        [edge_index,
         jnp.asarray(np.stack([ar % _CHUNK, ar % (_NPAD - _N) + _N]))],
        axis=1)
    zeros = jnp.asarray(np.zeros((_NPAD, _D), np.float32))

    h = x
    for i in (1, 2, 3):
        parts = _get_sc_segment_sum(h.shape[0])(edges, h, zeros)
        args = (h, parts,
                params[f'W{i}_1'], params[f'b{i}_1'],
                params[f'mlp_g{i}'], params[f'mlp_b{i}'],
                params[f'W{i}_2'], params[f'b{i}_2'])
        if i == 1:
            h = _mlp_first(*args, params[f'bn_g{i}'], params[f'bn_b{i}'])
        elif i == 2:
            h = _mlp_mid(*args, params[f'bn_g{i}'], params[f'bn_b{i}'])
        else:
            h = _mlp_final(*args)
    return h
